# baseline (speedup 1.0000x reference)
import functools

import jax
import jax.numpy as jnp
from jax import lax
from jax.experimental import pallas as pl
from jax.experimental.pallas import tpu as pltpu
from jax.experimental.pallas import tpu_sc as plsc

_NC = 2
_NS = 16
_NW = _NC * _NS
_L = 16
_GSZ = 128
_C0 = 120
_C1 = 40
_NROWS_USED = _NS * (_C0 + _C1)
_NROWS = _NROWS_USED + 48
_D = 128
_NPAD = 10240
_RPT = _NPAD // _NS
_BLK = 1024


def _sc_mesh():
    return plsc.VectorSubcoreMesh(core_axis_name="c", subcore_axis_name="s")


def _sc_degree(dst2):
    gsz = dst2.shape[1]
    nch = _NROWS_USED // _NW

    @functools.partial(
        pl.kernel,
        mesh=_sc_mesh(),
        out_type=jax.ShapeDtypeStruct((_NW, _NPAD), jnp.float32),
        scratch_types=[
            pltpu.VMEM((nch, gsz), jnp.int32),
            pltpu.VMEM((_NPAD,), jnp.float32),
        ],
        compiler_params=pltpu.CompilerParams(needs_layout_passes=False),
    )
    def k(dst_hbm, out_hbm, dst_v, hist_v):
        cid = lax.axis_index("c")
        sid = lax.axis_index("s")
        wid = cid * _NS + sid
        pltpu.sync_copy(dst_hbm.at[pl.ds(wid * nch, nch)], dst_v)

        z16 = jnp.zeros((_L,), jnp.float32)

        def zstep(i, c):
            hist_v[pl.ds(i * _L, _L)] = z16
            return c

        lax.fori_loop(0, _NPAD // _L, zstep, 0)

        ones16 = jnp.ones((_L,), jnp.float32)

        def estep(j, c):
            for k8 in range(gsz // _L):
                idx = dst_v[j, pl.ds(k8 * _L, _L)]
                plsc.addupdate_scatter(hist_v, [idx], ones16)
            return c

        lax.fori_loop(0, nch, estep, 0)
        pltpu.sync_copy(hist_v, out_hbm.at[wid])

    return k(dst2)


def _sc_aggregate(src2, dst2, y):
    gsz = src2.shape[1]
    nbuf = 2
    nphase = 5
    hcb = _C0 // nphase

    @functools.partial(
        pl.kernel,
        mesh=_sc_mesh(),
        out_type=jax.ShapeDtypeStruct((_NC, _NPAD, _D), jnp.float32),
        scratch_types=[
            pltpu.VMEM((hcb, gsz), jnp.int32),
            pltpu.VMEM((hcb, gsz), jnp.int32),
            [pltpu.VMEM((gsz, _D), jnp.float32)] * nbuf,
            pltpu.VMEM_SHARED((_NPAD, _D), jnp.float32),
            [pltpu.SemaphoreType.DMA] * nbuf,
            [pltpu.SemaphoreType.DMA] * nbuf,
        ],
    )
    def k(src_hbm, dst_hbm, y_hbm, out_hbm, src_v, dst_v, bufs, acc_sh,
          gsems, ssems):
        cid = lax.axis_index("c")
        sid = lax.axis_index("s")
        is0 = cid == 0
        row0 = jnp.where(is0, sid * _C0, _NS * _C0 + sid * _C1)
        hcc = jnp.where(is0, _C0 // nphase, _C1 // nphase)
        iters = jnp.where(is0, _C0 // nphase // nbuf,
                          _C1 // nphase // nbuf)

        z16 = jnp.zeros((_L,), jnp.float32)

        def zstep(i, c):
            for k8 in range(_D // _L):
                bufs[0][i, pl.ds(k8 * _L, _L)] = z16
            return c

        base = sid * _RPT
        lax.fori_loop(0, gsz, zstep, 0)
        for b in range(_RPT // gsz):
            pltpu.sync_copy(bufs[0], acc_sh.at[pl.ds(base + b * gsz, gsz)])
        plsc.subcore_barrier()

        def fire_gather(j, b):
            pltpu.async_copy(y_hbm.at[src_v.at[j]], bufs[b], gsems[b])

        def wait_gather(j, b):
            pltpu.make_async_copy(y_hbm.at[src_v.at[j]], bufs[b],
                                  gsems[b]).wait()

        for phase in range(nphase):
            pstart = row0 + phase * hcc
            pltpu.sync_copy(src_hbm.at[pl.ds(pstart, hcb)], src_v)
            pltpu.sync_copy(dst_hbm.at[pl.ds(pstart, hcb)], dst_v)

            @pl.when(iters > 0)
            def _():
                for b in range(nbuf):
                    fire_gather(b, b)

            def step(i, c):
                @pl.when(i < iters)
                def _():
                    j = i * nbuf
                    scat = []
                    for b in range(nbuf):
                        wait_gather(j + b, b)
                        scat.append(pltpu.async_copy(
                            bufs[b], acc_sh.at[dst_v.at[j + b]], ssems[b],
                            add=True))
                    for b in range(nbuf):
                        scat[b].wait()

                        @pl.when(j + b + nbuf < hcc)
                        def _():
                            fire_gather(j + b + nbuf, b)
                return c

            lax.fori_loop(0, _C0 // nphase // nbuf, step, 0)

        plsc.subcore_barrier()
        pltpu.sync_copy(acc_sh.at[pl.ds(base, _RPT)],
                        out_hbm.at[cid, pl.ds(base, _RPT)])

    return k(src2, dst2, y)


def _dinv_block(deg_ref):
    return lax.rsqrt(jnp.sum(deg_ref[...], axis=1, keepdims=True) + 1.0)


def _tc_y(degT, x, W):
    def body(deg_ref, x_ref, w_ref, y_ref):
        dinv = _dinv_block(deg_ref)
        y_ref[...] = dinv * jnp.dot(x_ref[...], w_ref[...],
                                    preferred_element_type=jnp.float32)

    return pl.pallas_call(
        body,
        grid=(_NPAD // _BLK,),
        in_specs=[
            pl.BlockSpec((_BLK, _NW), lambda i: (i, 0)),
            pl.BlockSpec((_BLK, _D), lambda i: (i, 0)),
            pl.BlockSpec((_D, _D), lambda i: (0, 0)),
        ],
        out_specs=pl.BlockSpec((_BLK, _D), lambda i: (i, 0)),
        out_shape=jax.ShapeDtypeStruct((_NPAD, _D), jnp.float32),
    )(degT, x, W)


def _tc_mid(degT, pa, pb, y, b, W):
    def body(deg_ref, pa_ref, pb_ref, y_ref, b_ref, w_ref, o_ref):
        dinv = _dinv_block(deg_ref)
        h = dinv * (pa_ref[...] + pb_ref[...] + y_ref[...]) + b_ref[...]
        o_ref[...] = dinv * jnp.dot(h, w_ref[...],
                                    preferred_element_type=jnp.float32)

    return pl.pallas_call(
        body,
        grid=(_NPAD // _BLK,),
        in_specs=[
            pl.BlockSpec((_BLK, _NW), lambda i: (i, 0)),
            pl.BlockSpec((_BLK, _D), lambda i: (i, 0)),
            pl.BlockSpec((_BLK, _D), lambda i: (i, 0)),
            pl.BlockSpec((_BLK, _D), lambda i: (i, 0)),
            pl.BlockSpec((1, _D), lambda i: (0, 0)),
            pl.BlockSpec((_D, _D), lambda i: (0, 0)),
        ],
        out_specs=pl.BlockSpec((_BLK, _D), lambda i: (i, 0)),
        out_shape=jax.ShapeDtypeStruct((_NPAD, _D), jnp.float32),
    )(degT, pa, pb, y, b, W)


def _tc_out(degT, pa, pb, y, b):
    def body(deg_ref, pa_ref, pb_ref, y_ref, b_ref, o_ref):
        dinv = _dinv_block(deg_ref)
        o_ref[...] = dinv * (pa_ref[...] + pb_ref[...] + y_ref[...]) + b_ref[...]

    return pl.pallas_call(
        body,
        grid=(_NPAD // _BLK,),
        in_specs=[
            pl.BlockSpec((_BLK, _NW), lambda i: (i, 0)),
            pl.BlockSpec((_BLK, _D), lambda i: (i, 0)),
            pl.BlockSpec((_BLK, _D), lambda i: (i, 0)),
            pl.BlockSpec((_BLK, _D), lambda i: (i, 0)),
            pl.BlockSpec((1, _D), lambda i: (0, 0)),
        ],
        out_specs=pl.BlockSpec((_BLK, _D), lambda i: (i, 0)),
        out_shape=jax.ShapeDtypeStruct((_NPAD, _D), jnp.float32),
    )(degT, pa, pb, y, b)


def kernel(feat, edge_index, W0, b0, W1, b1):
    n = feat.shape[0]
    e = edge_index.shape[1]
    epad = _NROWS * _GSZ
    src = jnp.concatenate(
        [edge_index[0], jnp.zeros((epad - e,), jnp.int32)]).reshape(
            _NROWS, _GSZ)
    dst = jnp.concatenate(
        [edge_index[1], jnp.full((epad - e,), n, jnp.int32)]).reshape(
            _NROWS, _GSZ)
    x = jnp.pad(feat, ((0, _NPAD - n), (0, 0)))

    deg_parts = _sc_degree(dst)
    degT = deg_parts.T

    y0 = _tc_y(degT, x, W0)
    p0 = _sc_aggregate(src, dst, y0)
    y1 = _tc_mid(degT, p0[0], p0[1], y0, b0.reshape(1, _D), W1)
    p1 = _sc_aggregate(src, dst, y1)
    out = _tc_out(degT, p1[0], p1[1], y1, b1.reshape(1, _D))
    return out[:n]


# speedup vs baseline: 1.0996x; 1.0996x over previous
import functools

import jax
import jax.numpy as jnp
from jax import lax
from jax.experimental import pallas as pl
from jax.experimental.pallas import tpu as pltpu
from jax.experimental.pallas import tpu_sc as plsc

_NC = 2
_NS = 16
_NW = _NC * _NS
_L = 16
_GSZ = 64
_C0 = 256
_C1 = 64
_NROWS_USED = _NS * (_C0 + _C1)
_NROWS = _NROWS_USED + 48
_D = 128
_NPAD = 10240
_RPT = _NPAD // _NS
_BLK = 1024


def _sc_mesh():
    return plsc.VectorSubcoreMesh(core_axis_name="c", subcore_axis_name="s")


def _sc_degree(dst2):
    gsz = dst2.shape[1]
    nch = _NROWS_USED // _NW

    @functools.partial(
        pl.kernel,
        mesh=_sc_mesh(),
        out_type=jax.ShapeDtypeStruct((_NW, _NPAD), jnp.float32),
        scratch_types=[
            pltpu.VMEM((nch, gsz), jnp.int32),
            pltpu.VMEM((_NPAD,), jnp.float32),
        ],
        compiler_params=pltpu.CompilerParams(needs_layout_passes=False),
    )
    def k(dst_hbm, out_hbm, dst_v, hist_v):
        cid = lax.axis_index("c")
        sid = lax.axis_index("s")
        wid = cid * _NS + sid
        pltpu.sync_copy(dst_hbm.at[pl.ds(wid * nch, nch)], dst_v)

        z16 = jnp.zeros((_L,), jnp.float32)

        def zstep(i, c):
            hist_v[pl.ds(i * _L, _L)] = z16
            return c

        lax.fori_loop(0, _NPAD // _L, zstep, 0)

        ones16 = jnp.ones((_L,), jnp.float32)

        def estep(j, c):
            for k8 in range(gsz // _L):
                idx = dst_v[j, pl.ds(k8 * _L, _L)]
                plsc.addupdate_scatter(hist_v, [idx], ones16)
            return c

        lax.fori_loop(0, nch, estep, 0)
        pltpu.sync_copy(hist_v, out_hbm.at[wid])

    return k(dst2)


def _sc_aggregate(src2, dst2, y):
    gsz = src2.shape[1]
    nbuf = 4
    nphase = 8
    hcb = _C0 // nphase

    @functools.partial(
        pl.kernel,
        mesh=_sc_mesh(),
        out_type=jax.ShapeDtypeStruct((_NC, _NPAD, _D), jnp.float32),
        scratch_types=[
            pltpu.VMEM((hcb, gsz), jnp.int32),
            pltpu.VMEM((hcb, gsz), jnp.int32),
            [pltpu.VMEM((gsz, _D), jnp.float32)] * nbuf,
            pltpu.VMEM_SHARED((_NPAD, _D), jnp.float32),
            [pltpu.SemaphoreType.DMA] * nbuf,
            [pltpu.SemaphoreType.DMA] * nbuf,
        ],
    )
    def k(src_hbm, dst_hbm, y_hbm, out_hbm, src_v, dst_v, bufs, acc_sh,
          gsems, ssems):
        cid = lax.axis_index("c")
        sid = lax.axis_index("s")
        is0 = cid == 0
        row0 = jnp.where(is0, sid * _C0, _NS * _C0 + sid * _C1)
        hcc = jnp.where(is0, _C0 // nphase, _C1 // nphase)
        iters = jnp.where(is0, _C0 // nphase // nbuf,
                          _C1 // nphase // nbuf)

        z16 = jnp.zeros((_L,), jnp.float32)

        def zstep(i, c):
            for k8 in range(_D // _L):
                bufs[0][i, pl.ds(k8 * _L, _L)] = z16
            return c

        base = sid * _RPT
        lax.fori_loop(0, gsz, zstep, 0)
        for b in range(_RPT // gsz):
            pltpu.sync_copy(bufs[0], acc_sh.at[pl.ds(base + b * gsz, gsz)])
        plsc.subcore_barrier()

        def fire_gather(j, b):
            pltpu.async_copy(y_hbm.at[src_v.at[j]], bufs[b], gsems[b])

        def wait_gather(j, b):
            pltpu.make_async_copy(y_hbm.at[src_v.at[j]], bufs[b],
                                  gsems[b]).wait()

        for phase in range(nphase):
            pstart = row0 + phase * hcc
            pltpu.sync_copy(src_hbm.at[pl.ds(pstart, hcb)], src_v)
            pltpu.sync_copy(dst_hbm.at[pl.ds(pstart, hcb)], dst_v)

            @pl.when(iters > 0)
            def _():
                for b in range(nbuf):
                    fire_gather(b, b)

            def step(i, c):
                @pl.when(i < iters)
                def _():
                    j = i * nbuf
                    scat = []
                    for b in range(nbuf):
                        wait_gather(j + b, b)
                        scat.append(pltpu.async_copy(
                            bufs[b], acc_sh.at[dst_v.at[j + b]], ssems[b],
                            add=True))
                    for b in range(nbuf):
                        scat[b].wait()

                        @pl.when(j + b + nbuf < hcc)
                        def _():
                            fire_gather(j + b + nbuf, b)
                return c

            lax.fori_loop(0, _C0 // nphase // nbuf, step, 0)

        plsc.subcore_barrier()
        pltpu.sync_copy(acc_sh.at[pl.ds(base, _RPT)],
                        out_hbm.at[cid, pl.ds(base, _RPT)])

    return k(src2, dst2, y)


def _dinv_block(deg_ref):
    return lax.rsqrt(jnp.sum(deg_ref[...], axis=1, keepdims=True) + 1.0)


def _tc_y(degT, x, W):
    def body(deg_ref, x_ref, w_ref, y_ref):
        dinv = _dinv_block(deg_ref)
        y_ref[...] = dinv * jnp.dot(x_ref[...], w_ref[...],
                                    preferred_element_type=jnp.float32)

    return pl.pallas_call(
        body,
        grid=(_NPAD // _BLK,),
        in_specs=[
            pl.BlockSpec((_BLK, _NW), lambda i: (i, 0)),
            pl.BlockSpec((_BLK, _D), lambda i: (i, 0)),
            pl.BlockSpec((_D, _D), lambda i: (0, 0)),
        ],
        out_specs=pl.BlockSpec((_BLK, _D), lambda i: (i, 0)),
        out_shape=jax.ShapeDtypeStruct((_NPAD, _D), jnp.float32),
    )(degT, x, W)


def _tc_mid(degT, pa, pb, y, b, W):
    def body(deg_ref, pa_ref, pb_ref, y_ref, b_ref, w_ref, o_ref):
        dinv = _dinv_block(deg_ref)
        h = dinv * (pa_ref[...] + pb_ref[...] + y_ref[...]) + b_ref[...]
        o_ref[...] = dinv * jnp.dot(h, w_ref[...],
                                    preferred_element_type=jnp.float32)

    return pl.pallas_call(
        body,
        grid=(_NPAD // _BLK,),
        in_specs=[
            pl.BlockSpec((_BLK, _NW), lambda i: (i, 0)),
            pl.BlockSpec((_BLK, _D), lambda i: (i, 0)),
            pl.BlockSpec((_BLK, _D), lambda i: (i, 0)),
            pl.BlockSpec((_BLK, _D), lambda i: (i, 0)),
            pl.BlockSpec((1, _D), lambda i: (0, 0)),
            pl.BlockSpec((_D, _D), lambda i: (0, 0)),
        ],
        out_specs=pl.BlockSpec((_BLK, _D), lambda i: (i, 0)),
        out_shape=jax.ShapeDtypeStruct((_NPAD, _D), jnp.float32),
    )(degT, pa, pb, y, b, W)


def _tc_out(degT, pa, pb, y, b):
    def body(deg_ref, pa_ref, pb_ref, y_ref, b_ref, o_ref):
        dinv = _dinv_block(deg_ref)
        o_ref[...] = dinv * (pa_ref[...] + pb_ref[...] + y_ref[...]) + b_ref[...]

    return pl.pallas_call(
        body,
        grid=(_NPAD // _BLK,),
        in_specs=[
            pl.BlockSpec((_BLK, _NW), lambda i: (i, 0)),
            pl.BlockSpec((_BLK, _D), lambda i: (i, 0)),
            pl.BlockSpec((_BLK, _D), lambda i: (i, 0)),
            pl.BlockSpec((_BLK, _D), lambda i: (i, 0)),
            pl.BlockSpec((1, _D), lambda i: (0, 0)),
        ],
        out_specs=pl.BlockSpec((_BLK, _D), lambda i: (i, 0)),
        out_shape=jax.ShapeDtypeStruct((_NPAD, _D), jnp.float32),
    )(degT, pa, pb, y, b)


def kernel(feat, edge_index, W0, b0, W1, b1):
    n = feat.shape[0]
    e = edge_index.shape[1]
    epad = _NROWS * _GSZ
    src = jnp.concatenate(
        [edge_index[0], jnp.zeros((epad - e,), jnp.int32)]).reshape(
            _NROWS, _GSZ)
    dst = jnp.concatenate(
        [edge_index[1], jnp.full((epad - e,), n, jnp.int32)]).reshape(
            _NROWS, _GSZ)
    x = jnp.pad(feat, ((0, _NPAD - n), (0, 0)))

    deg_parts = _sc_degree(dst)
    degT = deg_parts.T

    y0 = _tc_y(degT, x, W0)
    p0 = _sc_aggregate(src, dst, y0)
    y1 = _tc_mid(degT, p0[0], p0[1], y0, b0.reshape(1, _D), W1)
    p1 = _sc_aggregate(src, dst, y1)
    out = _tc_out(degT, p1[0], p1[1], y1, b1.reshape(1, _D))
    return out[:n]
